# baseline (speedup 1.0000x reference)
import functools

import jax
import jax.numpy as jnp
from jax.experimental import pallas as pl

S = 20
N = 128
HID = 32


def _to_nn(x):
    return jnp.swapaxes(x.reshape(N, N, 1), 1, 2).reshape(N, N)


def _to_pairs(x):
    return jnp.swapaxes(x.reshape(N, 1, N), 1, 2).reshape(N * N, 1)


def _layer_norm(x, w, b, eps=1e-05):
    u = jnp.mean(x, axis=-1, keepdims=True)
    s = jnp.mean((x - u) ** 2, axis=-1, keepdims=True)
    return w * (x - u) / jnp.sqrt(s + eps) + b


def _body(ci_ref, ni_ref, hs_ref, dc_ref, av_ref,
          relW_ref, relb_ref, rellnw_ref, rellnb_ref,
          ngW_ref, ngb_ref, nglnw_ref, nglnb_ref,
          warW_ref, warb_ref,
          wW_ref, wb_ref, wlnw_ref, wlnb_ref,
          out_ref):
    n = N
    d = HID
    p = n * n

    h_s = hs_ref[0]
    ci = ci_ref[...].reshape(p, 2)
    dc = dc_ref[0].reshape(p, 2)
    av = jnp.broadcast_to(av_ref[...][None, :, :], (n, n, 2)).reshape(p, 2)
    corr = jnp.concatenate([ci, dc, av], axis=-1)

    r_lin = jnp.dot(corr, relW_ref[...],
                    preferred_element_type=jnp.float32) + relb_ref[...]
    r_t = jax.nn.relu(_layer_norm(r_lin, rellnw_ref[...], rellnb_ref[...]))

    hi_t = jnp.broadcast_to(h_s[:, None, :], (n, n, d)).reshape(p, d)
    nei = jnp.broadcast_to(h_s[None, :, :], (n, n, d)).reshape(p, d)
    tmp = jnp.concatenate([r_t, hi_t, nei], axis=-1)

    ng_lin = jnp.dot(tmp, ngW_ref[...],
                     preferred_element_type=jnp.float32) + ngb_ref[...]
    nGate = jax.nn.sigmoid(_layer_norm(ng_lin, nglnw_ref[...], nglnb_ref[...]))

    tt = jax.nn.relu(jnp.dot(tmp, warW_ref[...],
                             preferred_element_type=jnp.float32)
                     + warb_ref[...])
    tt_nn = _to_nn(tt)

    mask = ni_ref[...] > 0
    pos_in = jnp.where(mask & (tt_nn != 0.0), tt_nn, -10000.0)
    pos = jax.nn.softmax(pos_in, axis=-1)
    coef = _to_pairs(jnp.where(mask, pos, 0.0))

    hm = nei * nGate * coef
    h_sum_in = hm.reshape(n, n, d).sum(axis=1)

    w_lin = jnp.dot(h_sum_in, wW_ref[...],
                    preferred_element_type=jnp.float32) + wb_ref[...]
    out_ref[0] = h_s + jax.nn.relu(_layer_norm(w_lin, wlnw_ref[...], wlnb_ref[...]))


@functools.partial(jax.jit, static_argnames=())
def _run(corr_index, nei_index, hidden_state, dest_corr, agent_v,
         rel_W, rel_b, rel_ln_w, rel_ln_b,
         ng_W, ng_b, ng_ln_w, ng_ln_b,
         war_W, war_b, w_W, w_b, w_ln_w, w_ln_b):
    full = lambda shape: pl.BlockSpec(shape, lambda s: (0,) * len(shape))
    grid_spec = pl.GridSpec(
        grid=(S,),
        in_specs=[
            full((N, N, 2)),
            full((N, N)),
            pl.BlockSpec((1, N, HID), lambda s: (s, 0, 0)),
            pl.BlockSpec((1, N, N, 2), lambda s: (s, 0, 0, 0)),
            full((N, 2)),
            full((6, HID)), full((HID,)), full((HID,)), full((HID,)),
            full((3 * HID, HID)), full((HID,)), full((HID,)), full((HID,)),
            full((3 * HID, 1)), full((1,)),
            full((HID, HID)), full((HID,)), full((HID,)), full((HID,)),
        ],
        out_specs=pl.BlockSpec((1, N, HID), lambda s: (s, 0, 0)),
    )
    return pl.pallas_call(
        _body,
        grid_spec=grid_spec,
        out_shape=jax.ShapeDtypeStruct((S, N, HID), jnp.float32),
    )(corr_index, nei_index, hidden_state, dest_corr, agent_v,
      rel_W, rel_b, rel_ln_w, rel_ln_b,
      ng_W, ng_b, ng_ln_w, ng_ln_b,
      war_W, war_b, w_W, w_b, w_ln_w, w_ln_b)


def kernel(corr_index, nei_index, nei_num, hidden_state, dest_corr, past_dest,
           agent_v, rel_W, rel_b, rel_ln_w, rel_ln_b, ng_W, ng_b, ng_ln_w,
           ng_ln_b, war_W, war_b, w_W, w_b, w_ln_w, w_ln_b):
    del nei_num, past_dest
    return _run(corr_index, nei_index, hidden_state, dest_corr, agent_v,
                rel_W, rel_b, rel_ln_w, rel_ln_b,
                ng_W, ng_b, ng_ln_w, ng_ln_b,
                war_W, war_b, w_W, w_b, w_ln_w, w_ln_b)


# speedup vs baseline: 1.1319x; 1.1319x over previous
import functools

import jax
import jax.numpy as jnp
from jax.experimental import pallas as pl

S = 20
N = 128
HID = 32


def _to_nn(x):
    return jnp.swapaxes(x.reshape(N, N, 1), 1, 2).reshape(N, N)


def _to_pairs(x):
    return jnp.swapaxes(x.reshape(N, 1, N), 1, 2).reshape(N * N, 1)


def _ln(x, w, b, eps=1e-05):
    m = jnp.full((HID, HID), 1.0 / HID, dtype=jnp.float32)
    u = jnp.dot(x, m, preferred_element_type=jnp.float32)
    d = x - u
    s = jnp.dot(d * d, m, preferred_element_type=jnp.float32)
    return d * (w * jax.lax.rsqrt(s + eps)) + b


def _body(corr_ref, ni_ref, hs_ref, dc_ref,
          relW4_ref, relW2_ref, relb_ref, rellnw_ref, rellnb_ref,
          ngW_ref, ngb_ref, nglnw_ref, nglnb_ref,
          warW_ref, warb_ref,
          wW_ref, wb_ref, wlnw_ref, wlnb_ref,
          out_ref):
    n = N
    d = HID
    p = n * n

    h_s = hs_ref[0]

    r_lin = (jnp.dot(corr_ref[...], relW4_ref[...],
                     preferred_element_type=jnp.float32)
             + jnp.dot(dc_ref[0], relW2_ref[...],
                       preferred_element_type=jnp.float32)
             + relb_ref[...])
    r_t = jax.nn.relu(_ln(r_lin, rellnw_ref[...], rellnb_ref[...]))

    hi_t = jnp.broadcast_to(h_s[:, None, :], (n, n, d)).reshape(p, d)
    nei = jnp.broadcast_to(h_s[None, :, :], (n, n, d)).reshape(p, d)
    tmp = jnp.concatenate([r_t, hi_t, nei], axis=-1)

    ng_lin = jnp.dot(tmp, ngW_ref[...],
                     preferred_element_type=jnp.float32) + ngb_ref[...]
    nGate = jax.nn.sigmoid(_ln(ng_lin, nglnw_ref[...], nglnb_ref[...]))

    tt = jax.nn.relu(jnp.dot(tmp, warW_ref[...],
                             preferred_element_type=jnp.float32)
                     + warb_ref[...])
    tt_nn = _to_nn(tt)

    mask = ni_ref[...] > 0
    pos_in = jnp.where(mask & (tt_nn != 0.0), tt_nn, -10000.0)
    pos = jax.nn.softmax(pos_in, axis=-1)
    coef = _to_pairs(jnp.where(mask, pos, 0.0))

    hm = nei * nGate * coef
    h_sum_in = hm.reshape(n, n, d).sum(axis=1)

    w_lin = jnp.dot(h_sum_in, wW_ref[...],
                    preferred_element_type=jnp.float32) + wb_ref[...]
    out_ref[0] = h_s + jax.nn.relu(_ln(w_lin, wlnw_ref[...], wlnb_ref[...]))


@functools.partial(jax.jit, static_argnames=())
def _run(corr_index, nei_index, hidden_state, dest_corr, agent_v,
         rel_W, rel_b, rel_ln_w, rel_ln_b,
         ng_W, ng_b, ng_ln_w, ng_ln_b,
         war_W, war_b, w_W, w_b, w_ln_w, w_ln_b):
    p = N * N
    corr4 = jnp.concatenate(
        [corr_index.reshape(p, 2), jnp.tile(agent_v, (N, 1))], axis=-1)
    relW4 = jnp.concatenate([rel_W[0:2], rel_W[4:6]], axis=0)
    relW2 = rel_W[2:4]
    dc = dest_corr.reshape(S, p, 2)

    full = lambda shape: pl.BlockSpec(shape, lambda s: (0,) * len(shape))
    grid_spec = pl.GridSpec(
        grid=(S,),
        in_specs=[
            full((p, 4)),
            full((N, N)),
            pl.BlockSpec((1, N, HID), lambda s: (s, 0, 0)),
            pl.BlockSpec((1, p, 2), lambda s: (s, 0, 0)),
            full((4, HID)), full((2, HID)),
            full((HID,)), full((HID,)), full((HID,)),
            full((3 * HID, HID)), full((HID,)), full((HID,)), full((HID,)),
            full((3 * HID, 1)), full((1,)),
            full((HID, HID)), full((HID,)), full((HID,)), full((HID,)),
        ],
        out_specs=pl.BlockSpec((1, N, HID), lambda s: (s, 0, 0)),
    )
    return pl.pallas_call(
        _body,
        grid_spec=grid_spec,
        out_shape=jax.ShapeDtypeStruct((S, N, HID), jnp.float32),
    )(corr4, nei_index, hidden_state, dc,
      relW4, relW2, rel_b, rel_ln_w, rel_ln_b,
      ng_W, ng_b, ng_ln_w, ng_ln_b,
      war_W, war_b, w_W, w_b, w_ln_w, w_ln_b)


def kernel(corr_index, nei_index, nei_num, hidden_state, dest_corr, past_dest,
           agent_v, rel_W, rel_b, rel_ln_w, rel_ln_b, ng_W, ng_b, ng_ln_w,
           ng_ln_b, war_W, war_b, w_W, w_b, w_ln_w, w_ln_b):
    del nei_num, past_dest
    return _run(corr_index, nei_index, hidden_state, dest_corr, agent_v,
                rel_W, rel_b, rel_ln_w, rel_ln_b,
                ng_W, ng_b, ng_ln_w, ng_ln_b,
                war_W, war_b, w_W, w_b, w_ln_w, w_ln_b)


# speedup vs baseline: 1.1337x; 1.0016x over previous
import functools

import jax
import jax.numpy as jnp
from jax.experimental import pallas as pl
from jax.experimental.pallas import tpu as pltpu

S = 20
N = 128
HID = 32


def _to_nn(x):
    return jnp.swapaxes(x.reshape(N, N, 1), 1, 2).reshape(N, N)


def _to_pairs(x):
    return jnp.swapaxes(x.reshape(N, 1, N), 1, 2).reshape(N * N, 1)


def _ln(x, w, b, eps=1e-05):
    m = jnp.full((HID, HID), 1.0 / HID, dtype=jnp.float32)
    u = jnp.dot(x, m, preferred_element_type=jnp.float32)
    d = x - u
    s = jnp.dot(d * d, m, preferred_element_type=jnp.float32)
    return d * (w * jax.lax.rsqrt(s + eps)) + b


def _body(corr_ref, ni_ref, hs_ref, dc_ref,
          relW4_ref, relW2_ref, relb_ref, rellnw_ref, rellnb_ref,
          ngW_ref, ngb_ref, nglnw_ref, nglnb_ref,
          warW_ref, warb_ref,
          wW_ref, wb_ref, wlnw_ref, wlnb_ref,
          out_ref):
    n = N
    d = HID
    p = n * n

    h_s = hs_ref[0]

    r_lin = (jnp.dot(corr_ref[...], relW4_ref[...],
                     preferred_element_type=jnp.float32)
             + jnp.dot(dc_ref[0], relW2_ref[...],
                       preferred_element_type=jnp.float32)
             + relb_ref[...])
    r_t = jax.nn.relu(_ln(r_lin, rellnw_ref[...], rellnb_ref[...]))

    hi_t = jnp.broadcast_to(h_s[:, None, :], (n, n, d)).reshape(p, d)
    nei = jnp.broadcast_to(h_s[None, :, :], (n, n, d)).reshape(p, d)
    tmp = jnp.concatenate([r_t, hi_t, nei], axis=-1)

    ng_lin = jnp.dot(tmp, ngW_ref[...],
                     preferred_element_type=jnp.float32) + ngb_ref[...]
    nGate = jax.nn.sigmoid(_ln(ng_lin, nglnw_ref[...], nglnb_ref[...]))

    tt = jax.nn.relu(jnp.dot(tmp, warW_ref[...],
                             preferred_element_type=jnp.float32)
                     + warb_ref[...])
    tt_nn = _to_nn(tt)

    mask = ni_ref[...] > 0
    pos_in = jnp.where(mask & (tt_nn != 0.0), tt_nn, -10000.0)
    pos = jax.nn.softmax(pos_in, axis=-1)
    coef = _to_pairs(jnp.where(mask, pos, 0.0))

    hm = nei * nGate * coef
    h_sum_in = hm.reshape(n, n, d).sum(axis=1)

    w_lin = jnp.dot(h_sum_in, wW_ref[...],
                    preferred_element_type=jnp.float32) + wb_ref[...]
    out_ref[0] = h_s + jax.nn.relu(_ln(w_lin, wlnw_ref[...], wlnb_ref[...]))


@functools.partial(jax.jit, static_argnames=())
def _run(corr_index, nei_index, hidden_state, dest_corr, agent_v,
         rel_W, rel_b, rel_ln_w, rel_ln_b,
         ng_W, ng_b, ng_ln_w, ng_ln_b,
         war_W, war_b, w_W, w_b, w_ln_w, w_ln_b):
    p = N * N
    corr4 = jnp.concatenate(
        [corr_index.reshape(p, 2), jnp.tile(agent_v, (N, 1))], axis=-1)
    relW4 = jnp.concatenate([rel_W[0:2], rel_W[4:6]], axis=0)
    relW2 = rel_W[2:4]
    dc = dest_corr.reshape(S, p, 2)

    full = lambda shape: pl.BlockSpec(shape, lambda s: (0,) * len(shape))
    grid_spec = pl.GridSpec(
        grid=(S,),
        in_specs=[
            full((p, 4)),
            full((N, N)),
            pl.BlockSpec((1, N, HID), lambda s: (s, 0, 0)),
            pl.BlockSpec((1, p, 2), lambda s: (s, 0, 0)),
            full((4, HID)), full((2, HID)),
            full((HID,)), full((HID,)), full((HID,)),
            full((3 * HID, HID)), full((HID,)), full((HID,)), full((HID,)),
            full((3 * HID, 1)), full((1,)),
            full((HID, HID)), full((HID,)), full((HID,)), full((HID,)),
        ],
        out_specs=pl.BlockSpec((1, N, HID), lambda s: (s, 0, 0)),
    )
    return pl.pallas_call(
        _body,
        grid_spec=grid_spec,
        compiler_params=pltpu.CompilerParams(
            dimension_semantics=("parallel",)),
        out_shape=jax.ShapeDtypeStruct((S, N, HID), jnp.float32),
    )(corr4, nei_index, hidden_state, dc,
      relW4, relW2, rel_b, rel_ln_w, rel_ln_b,
      ng_W, ng_b, ng_ln_w, ng_ln_b,
      war_W, war_b, w_W, w_b, w_ln_w, w_ln_b)


def kernel(corr_index, nei_index, nei_num, hidden_state, dest_corr, past_dest,
           agent_v, rel_W, rel_b, rel_ln_w, rel_ln_b, ng_W, ng_b, ng_ln_w,
           ng_ln_b, war_W, war_b, w_W, w_b, w_ln_w, w_ln_b):
    del nei_num, past_dest
    return _run(corr_index, nei_index, hidden_state, dest_corr, agent_v,
                rel_W, rel_b, rel_ln_w, rel_ln_b,
                ng_W, ng_b, ng_ln_w, ng_ln_b,
                war_W, war_b, w_W, w_b, w_ln_w, w_ln_b)


# speedup vs baseline: 2.4939x; 2.1998x over previous
import functools

import jax
import jax.numpy as jnp
from jax.experimental import pallas as pl
from jax.experimental.pallas import tpu as pltpu

S = 20
N = 128
HID = 32
G = 4
NB = N // G
P4 = N * NB


def _split4(x):
    return jnp.concatenate([x[0:32], x[32:64], x[64:96], x[96:128]], axis=1)


def _bc_rows(x):
    return jnp.broadcast_to(x[:, None, :], (N, NB, 128)).reshape(P4, 128)


def _bc_cols(x):
    return jnp.broadcast_to(x[None, :, :], (N, NB, 128)).reshape(P4, 128)


def _body(corr_ref, dc_ref, ni_ref, hs_ref, mblk_ref,
          w4blk_ref, w2blk_ref, relb_ref, rellnw_ref, rellnb_ref,
          ngWr_ref, ngWh_ref, ngWn_ref, ngb_ref, nglnw_ref, nglnb_ref,
          warWr_ref, warWh_ref, warWn_ref, warb_ref, rup_ref,
          wW_ref, wb_ref, wlnw_ref, wlnb_ref,
          out_ref):
    mblk = mblk_ref[...]

    def ln_packed(x, w, b, eps=1e-05):
        u = jnp.dot(x, mblk, preferred_element_type=jnp.float32)
        d = x - u
        s = jnp.dot(d * d, mblk, preferred_element_type=jnp.float32)
        return d * (w * jax.lax.rsqrt(s + eps)) + b

    h_s = hs_ref[0]

    r_lin = (jnp.dot(corr_ref[...], w4blk_ref[...],
                     preferred_element_type=jnp.float32)
             + jnp.dot(dc_ref[0], w2blk_ref[...],
                       preferred_element_type=jnp.float32)
             + relb_ref[...])
    r_t = jax.nn.relu(ln_packed(r_lin, rellnw_ref[...], rellnb_ref[...]))

    hh = jnp.dot(h_s, ngWh_ref[...], preferred_element_type=jnp.float32)
    hn = jnp.dot(h_s, ngWn_ref[...], preferred_element_type=jnp.float32)
    hh_bc = _bc_rows(jnp.concatenate([hh, hh, hh, hh], axis=1))
    hn_bc = _bc_cols(_split4(hn))
    ng_lin = (jnp.dot(r_t, ngWr_ref[...], preferred_element_type=jnp.float32)
              + hh_bc + hn_bc + ngb_ref[...])
    nGate = jax.nn.sigmoid(ln_packed(ng_lin, nglnw_ref[...], nglnb_ref[...]))

    wh = jnp.dot(h_s, warWh_ref[...], preferred_element_type=jnp.float32)
    wn = jnp.dot(h_s, warWn_ref[...], preferred_element_type=jnp.float32)
    z4 = (jnp.dot(r_t, warWr_ref[...], preferred_element_type=jnp.float32)
          .reshape(N, NB, G)
          + wh.reshape(N, 1, 1) + _split4(wn)[None, :, :] + warb_ref[...])
    tt3 = jnp.swapaxes(jax.nn.relu(z4), 1, 2)
    tt_nn = jnp.concatenate(
        [tt3[:, 0, :], tt3[:, 1, :], tt3[:, 2, :], tt3[:, 3, :]], axis=1)

    mask = ni_ref[...] > 0
    pos_in = jnp.where(mask & (tt_nn != 0.0), tt_nn, -10000.0)
    pos = jax.nn.softmax(pos_in, axis=-1)
    coef = jnp.where(mask, pos, 0.0)

    c34 = jnp.concatenate(
        [coef[:, 0:32].reshape(N, 1, NB), coef[:, 32:64].reshape(N, 1, NB),
         coef[:, 64:96].reshape(N, 1, NB), coef[:, 96:128].reshape(N, 1, NB)],
        axis=1)
    coef4 = jnp.swapaxes(c34, 1, 2).reshape(P4, G)
    coef_p = jnp.dot(coef4, rup_ref[...],
                     preferred_element_type=jnp.float32)

    nei_p = _bc_cols(_split4(h_s))
    hm = nei_p * nGate * coef_p
    hsum = hm.reshape(N, NB, 128).sum(axis=1)
    h_sum_in = (hsum[:, 0:32] + hsum[:, 32:64]
                + hsum[:, 64:96] + hsum[:, 96:128])

    w_lin = jnp.dot(h_sum_in, wW_ref[...],
                    preferred_element_type=jnp.float32) + wb_ref[...]
    m32 = jnp.full((HID, HID), 1.0 / HID, dtype=jnp.float32)
    u = jnp.dot(w_lin, m32, preferred_element_type=jnp.float32)
    d = w_lin - u
    sv = jnp.dot(d * d, m32, preferred_element_type=jnp.float32)
    ln_out = d * (wlnw_ref[...] * jax.lax.rsqrt(sv + 1e-05)) + wlnb_ref[...]
    out_ref[0] = h_s + jax.nn.relu(ln_out)


def _blkdiag4(w):
    return jax.scipy.linalg.block_diag(w, w, w, w)


@functools.partial(jax.jit, static_argnames=())
def _run(corr_index, nei_index, hidden_state, dest_corr, agent_v,
         rel_W, rel_b, rel_ln_w, rel_ln_b,
         ng_W, ng_b, ng_ln_w, ng_ln_b,
         war_W, war_b, w_W, w_b, w_ln_w, w_ln_b):
    p = N * N
    corr4 = jnp.concatenate(
        [corr_index.reshape(p, 2), jnp.tile(agent_v, (N, 1))], axis=-1)
    corr4_p = (corr4.reshape(N, G, NB, 4).transpose(0, 2, 1, 3)
               .reshape(P4, 4 * G))
    dc_p = (dest_corr.reshape(S, N, G, NB, 2).transpose(0, 1, 3, 2, 4)
            .reshape(S, P4, 2 * G))

    relW4 = jnp.concatenate([rel_W[0:2], rel_W[4:6]], axis=0)
    w4blk = _blkdiag4(relW4)
    w2blk = _blkdiag4(rel_W[2:4])
    mblk = _blkdiag4(jnp.full((HID, HID), 1.0 / HID, jnp.float32))
    ngWr_blk = _blkdiag4(ng_W[0:HID])
    warWr_blk = _blkdiag4(war_W[0:HID])
    rup = _blkdiag4(jnp.ones((1, HID), jnp.float32))

    t4 = lambda v: jnp.tile(v, G)
    full = lambda shape: pl.BlockSpec(shape, lambda s: (0,) * len(shape))
    grid_spec = pl.GridSpec(
        grid=(S,),
        in_specs=[
            full((P4, 4 * G)),
            pl.BlockSpec((1, P4, 2 * G), lambda s: (s, 0, 0)),
            full((N, N)),
            pl.BlockSpec((1, N, HID), lambda s: (s, 0, 0)),
            full((128, 128)),
            full((16, 128)), full((8, 128)),
            full((128,)), full((128,)), full((128,)),
            full((128, 128)), full((HID, HID)), full((HID, HID)),
            full((128,)), full((128,)), full((128,)),
            full((128, G)), full((HID, 1)), full((HID, 1)), full((1,)),
            full((G, 128)),
            full((HID, HID)), full((HID,)), full((HID,)), full((HID,)),
        ],
        out_specs=pl.BlockSpec((1, N, HID), lambda s: (s, 0, 0)),
    )
    return pl.pallas_call(
        _body,
        grid_spec=grid_spec,
        compiler_params=pltpu.CompilerParams(
            dimension_semantics=("parallel",)),
        out_shape=jax.ShapeDtypeStruct((S, N, HID), jnp.float32),
    )(corr4_p, dc_p, nei_index, hidden_state, mblk,
      w4blk, w2blk, t4(rel_b), t4(rel_ln_w), t4(rel_ln_b),
      ngWr_blk, ng_W[HID:2 * HID], ng_W[2 * HID:], t4(ng_b), t4(ng_ln_w),
      t4(ng_ln_b),
      warWr_blk, war_W[HID:2 * HID], war_W[2 * HID:], war_b, rup,
      w_W, w_b, w_ln_w, w_ln_b)


def kernel(corr_index, nei_index, nei_num, hidden_state, dest_corr, past_dest,
           agent_v, rel_W, rel_b, rel_ln_w, rel_ln_b, ng_W, ng_b, ng_ln_w,
           ng_ln_b, war_W, war_b, w_W, w_b, w_ln_w, w_ln_b):
    del nei_num, past_dest
    return _run(corr_index, nei_index, hidden_state, dest_corr, agent_v,
                rel_W, rel_b, rel_ln_w, rel_ln_b,
                ng_W, ng_b, ng_ln_w, ng_ln_b,
                war_W, war_b, w_W, w_b, w_ln_w, w_ln_b)
